# baseline (speedup 1.0000x reference)
import functools

import jax
import jax.numpy as jnp
from jax import lax
from jax.experimental import pallas as pl
from jax.experimental.pallas import tpu as pltpu
from jax.experimental.pallas import tpu_sc as plsc

N_NODES = 10000
N_EDGES = 320000
D_FEAT = 128
D_EDGE = 16
D_PACK = D_FEAT // 2

NC = 2
NS = 16
LANES = 16
NW = NC * NS
E_PER_W = N_EDGES // NW
CHUNK = 40
N_CHUNKS = E_PER_W // CHUNK
STRIPE = 640
ZCHUNK = 40
NZ = STRIPE // ZCHUNK

_WPERM = ([32 * (u // 16) + u % 16 for u in range(D_PACK)]
          + [32 * (u // 16) + 16 + u % 16 for u in range(D_PACK)])


def _pack_to(acc, o_ref):
    au = jax.lax.bitcast_convert_type(acc[:, :D_PACK], jnp.uint32)
    bu = jax.lax.bitcast_convert_type(acc[:, D_PACK:], jnp.uint32)
    ar = (au + 0x7FFF + ((au >> 16) & 1)) >> 16
    br = (bu + 0x7FFF + ((bu >> 16) & 1)) >> 16
    o_ref[...] = jax.lax.bitcast_convert_type(ar | (br << 16), jnp.float32)


def _mm_body(x_ref, w_ref, o_ref):
    o_ref[...] = jnp.dot(x_ref[...], w_ref[...], preferred_element_type=jnp.float32)


def _mm_bias_pack_body(x_ref, w_ref, b_ref, o_ref):
    acc = jnp.dot(x_ref[...], w_ref[...], preferred_element_type=jnp.float32)
    _pack_to(acc + b_ref[...], o_ref)


def _update_body(h_ref, z0_ref, z1_ref, w1_ref, w2_ref, b_ref, o_ref):
    z = z0_ref[...] + z1_ref[...]
    acc = jnp.dot(h_ref[...], w1_ref[...], preferred_element_type=jnp.float32)
    acc = acc + jnp.dot(z, w2_ref[...], preferred_element_type=jnp.float32)
    o_ref[...] = jnp.maximum(acc + b_ref[...], 0.0)


def _node_matmul(x, w, block_m):
    m, k = x.shape
    n = w.shape[1]
    return pl.pallas_call(
        _mm_body,
        grid=(m // block_m,),
        in_specs=[
            pl.BlockSpec((block_m, k), lambda i: (i, 0)),
            pl.BlockSpec((k, n), lambda i: (0, 0)),
        ],
        out_specs=pl.BlockSpec((block_m, n), lambda i: (i, 0)),
        out_shape=jax.ShapeDtypeStruct((m, n), jnp.float32),
    )(x, w)


def _edge_matmul(x, w, b, block_m):
    m, k = x.shape
    n = w.shape[1]
    return pl.pallas_call(
        _mm_bias_pack_body,
        grid=(m // block_m,),
        in_specs=[
            pl.BlockSpec((block_m, k), lambda i: (i, 0)),
            pl.BlockSpec((k, n), lambda i: (0, 0)),
            pl.BlockSpec((1, n), lambda i: (0, 0)),
        ],
        out_specs=pl.BlockSpec((block_m, n // 2), lambda i: (i, 0)),
        out_shape=jax.ShapeDtypeStruct((m, n // 2), jnp.float32),
    )(x, w, b)


def _node_update(h, z0, z1, w1, w2, b, block_m):
    m, k = h.shape
    n = w1.shape[1]
    return pl.pallas_call(
        _update_body,
        grid=(m // block_m,),
        in_specs=[
            pl.BlockSpec((block_m, k), lambda i: (i, 0)),
            pl.BlockSpec((block_m, k), lambda i: (i, 0)),
            pl.BlockSpec((block_m, k), lambda i: (i, 0)),
            pl.BlockSpec((k, n), lambda i: (0, 0)),
            pl.BlockSpec((k, n), lambda i: (0, 0)),
            pl.BlockSpec((1, n), lambda i: (0, 0)),
        ],
        out_specs=pl.BlockSpec((block_m, n), lambda i: (i, 0)),
        out_shape=jax.ShapeDtypeStruct((m, n), jnp.float32),
    )(h, z0, z1, w1, w2, b)


_MESH = plsc.VectorSubcoreMesh(core_axis_name="c", subcore_axis_name="s")


@functools.partial(
    pl.kernel,
    out_type=(
        jax.ShapeDtypeStruct((N_NODES, D_FEAT), jnp.float32),
        jax.ShapeDtypeStruct((N_NODES, D_FEAT), jnp.float32),
    ),
    mesh=_MESH,
    scratch_types=[
        pltpu.VMEM((E_PER_W,), jnp.int32),
        pltpu.VMEM((E_PER_W,), jnp.int32),
        pltpu.VMEM((CHUNK, D_FEAT), jnp.float32),
        pltpu.VMEM((CHUNK, D_FEAT), jnp.float32),
        pltpu.VMEM((CHUNK, D_PACK), jnp.float32),
        pltpu.VMEM((CHUNK, D_PACK), jnp.float32),
        pltpu.VMEM((CHUNK, D_FEAT), jnp.float32),
        pltpu.VMEM((CHUNK, D_FEAT), jnp.float32),
        pltpu.VMEM_SHARED((N_NODES, D_FEAT), jnp.float32),
        pltpu.SemaphoreType.DMA,
        pltpu.SemaphoreType.DMA,
        pltpu.SemaphoreType.DMA,
        pltpu.SemaphoreType.DMA,
        pltpu.SemaphoreType.DMA,
        pltpu.SemaphoreType.DMA,
    ],
)
def _edge_phase(hw_hbm, xew_hbm, src_hbm, dst_hbm, out0_hbm, out1_hbm,
                sidx_all, didx_all, rows0, rows1, xb0, xb1, y0, y1,
                z_sh, sg0, sg1, sx0, sx1, ss0, ss1):
    rows = (rows0, rows1)
    xb = (xb0, xb1)
    y = (y0, y1)
    sg = (sg0, sg1)
    sx = (sx0, sx1)
    ss = (ss0, ss1)

    cid = lax.axis_index("c")
    sid = lax.axis_index("s")
    wid = cid * NS + sid
    ebase = wid * E_PER_W

    pltpu.sync_copy(src_hbm.at[pl.ds(ebase, E_PER_W)], sidx_all)
    pltpu.sync_copy(dst_hbm.at[pl.ds(ebase, E_PER_W)], didx_all)

    zeros = jnp.zeros((LANES,), jnp.float32)

    def zero_row(i, carry):
        for j in range(D_FEAT // LANES):
            y0[i, pl.ds(j * LANES, LANES)] = zeros
        return carry

    lax.fori_loop(0, ZCHUNK, zero_row, 0)
    row0 = sid * STRIPE
    for k in range(NZ):
        r0 = row0 + k * ZCHUNK

        @pl.when(r0 + ZCHUNK <= N_NODES)
        def _():
            pltpu.sync_copy(y0, z_sh.at[pl.ds(r0, ZCHUNK)])

    plsc.subcore_barrier()

    def issue(b, c):
        eoff = ebase + c * CHUNK
        pltpu.async_copy(xew_hbm.at[pl.ds(eoff, CHUNK)], xb[b], sx[b])
        pltpu.async_copy(
            hw_hbm.at[sidx_all.at[pl.ds(c * CHUNK, CHUNK)]], rows[b], sg[b])

    def wait_inputs(b, c):
        pltpu.make_async_copy(xew_hbm.at[pl.ds(ebase, CHUNK)], xb[b], sx[b]).wait()
        pltpu.make_async_copy(
            hw_hbm.at[sidx_all.at[pl.ds(c * CHUNK, CHUNK)]], rows[b], sg[b]).wait()

    def scatter_ref(c):
        return z_sh.at[didx_all.at[pl.ds(c * CHUNK, CHUNK)]]

    mask_hi = jnp.uint32(0xFFFF0000)

    def compute(b):
        def row_body(r, carry):
            for j in range(D_PACK // LANES):
                u = jax.lax.bitcast_convert_type(
                    xb[b][r, pl.ds(j * LANES, LANES)], jnp.uint32)
                sl_lo = pl.ds(2 * j * LANES, LANES)
                sl_hi = pl.ds((2 * j + 1) * LANES, LANES)
                lo = jax.lax.bitcast_convert_type(u << 16, jnp.float32)
                hi = jax.lax.bitcast_convert_type(u & mask_hi, jnp.float32)
                y[b][r, sl_lo] = jnp.maximum(rows[b][r, sl_lo] + lo, 0.0)
                y[b][r, sl_hi] = jnp.maximum(rows[b][r, sl_hi] + hi, 0.0)
            return carry

        lax.fori_loop(0, CHUNK, row_body, 0)

    issue(0, 0)
    issue(1, 1)

    def pair_body(o, carry):
        for b in range(2):
            c = 2 * o + b
            wait_inputs(b, c)

            @pl.when(o > 0)
            def _():
                pltpu.make_async_copy(y[b], scatter_ref(c - 2), ss[b]).wait()

            compute(b)
            pltpu.async_copy(y[b], scatter_ref(c), ss[b], add=True)

            @pl.when(c + 2 < N_CHUNKS)
            def _():
                issue(b, c + 2)

        return carry

    lax.fori_loop(0, N_CHUNKS // 2, pair_body, 0)

    pltpu.make_async_copy(y[0], scatter_ref(N_CHUNKS - 2), ss[0]).wait()
    pltpu.make_async_copy(y[1], scatter_ref(N_CHUNKS - 1), ss[1]).wait()
    plsc.subcore_barrier()

    for k in range(NZ):
        r0 = row0 + k * ZCHUNK

        @pl.when(r0 + ZCHUNK <= N_NODES)
        def _():
            pltpu.sync_copy(z_sh.at[pl.ds(r0, ZCHUNK)], y0)

            @pl.when(cid == 0)
            def _():
                pltpu.sync_copy(y0, out0_hbm.at[pl.ds(r0, ZCHUNK)])

            @pl.when(cid == 1)
            def _():
                pltpu.sync_copy(y0, out1_hbm.at[pl.ds(r0, ZCHUNK)])


@jax.jit
def kernel(H, Xe, id_Xe, W_M, b_M, W_U, b_U):
    src = id_Xe[0].astype(jnp.int32)
    dst = id_Xe[1].astype(jnp.int32)
    perm = jnp.asarray(_WPERM, dtype=jnp.int32)
    wm2 = W_M[D_FEAT:, perm]
    bm = b_M[perm]
    hw = _node_matmul(H, W_M[:D_FEAT], block_m=2000)
    xewb = _edge_matmul(Xe, wm2, bm.reshape(1, -1), block_m=4000)
    z0, z1 = _edge_phase(hw, xewb, src, dst)
    return _node_update(H, z0, z1, W_U[:D_FEAT], W_U[D_FEAT:],
                        b_U.reshape(1, -1), block_m=2000)


# speedup vs baseline: 1.5419x; 1.5419x over previous
import functools

import jax
import jax.numpy as jnp
from jax import lax
from jax.experimental import pallas as pl
from jax.experimental.pallas import tpu as pltpu
from jax.experimental.pallas import tpu_sc as plsc

N_NODES = 10000
N_EDGES = 320000
D_FEAT = 128
D_EDGE = 16
D_PACK = D_FEAT // 2

NC = 2
NS = 16
LANES = 16
NW = NC * NS
E_PER_W = N_EDGES // NW
CHUNK = 40
N_CHUNKS = E_PER_W // CHUNK
STRIPE = 640
ZCHUNK = 40
NZ = STRIPE // ZCHUNK

_WPERM = ([32 * (u // 16) + u % 16 for u in range(D_PACK)]
          + [32 * (u // 16) + 16 + u % 16 for u in range(D_PACK)])


def _pack_to(acc, o_ref):
    n = acc.shape[1] // 2
    au = jax.lax.bitcast_convert_type(acc[:, :n], jnp.uint32)
    bu = jax.lax.bitcast_convert_type(acc[:, n:], jnp.uint32)
    ar = (au + 0x7FFF + ((au >> 16) & 1)) >> 16
    br = (bu + 0x7FFF + ((bu >> 16) & 1)) >> 16
    o_ref[...] = jax.lax.bitcast_convert_type(ar | (br << 16), jnp.float32)


def _mm_body(x_ref, w_ref, o_ref):
    o_ref[...] = jnp.dot(x_ref[...], w_ref[...], preferred_element_type=jnp.float32)


def _mm_bias_pack_body(x_ref, w_ref, b_ref, o_ref):
    acc = jnp.dot(x_ref[...], w_ref[...], preferred_element_type=jnp.float32)
    _pack_to(acc + b_ref[...], o_ref)


def _update_body(h_ref, z0_ref, z1_ref, w1_ref, w2_ref, b_ref, o_ref):
    z = z0_ref[...] + z1_ref[...]
    acc = jnp.dot(h_ref[...], w1_ref[...], preferred_element_type=jnp.float32)
    acc = acc + jnp.dot(z, w2_ref[...], preferred_element_type=jnp.float32)
    o_ref[...] = jnp.maximum(acc + b_ref[...], 0.0)


def _node_matmul(x, w, block_m):
    m, k = x.shape
    n = w.shape[1]
    return pl.pallas_call(
        _mm_body,
        grid=(m // block_m,),
        in_specs=[
            pl.BlockSpec((block_m, k), lambda i: (i, 0)),
            pl.BlockSpec((k, n), lambda i: (0, 0)),
        ],
        out_specs=pl.BlockSpec((block_m, n), lambda i: (i, 0)),
        out_shape=jax.ShapeDtypeStruct((m, n), jnp.float32),
    )(x, w)


def _edge_matmul(x, w, b, block_m):
    m, k = x.shape
    n = w.shape[1]
    return pl.pallas_call(
        _mm_bias_pack_body,
        grid=(m // block_m,),
        in_specs=[
            pl.BlockSpec((block_m, k), lambda i: (i, 0)),
            pl.BlockSpec((k, n), lambda i: (0, 0)),
            pl.BlockSpec((1, n), lambda i: (0, 0)),
        ],
        out_specs=pl.BlockSpec((block_m, n // 2), lambda i: (i, 0)),
        out_shape=jax.ShapeDtypeStruct((m, n // 2), jnp.float32),
    )(x, w, b)


def _node_update(h, z0, z1, w1, w2, b, block_m):
    m, k = h.shape
    n = w1.shape[1]
    return pl.pallas_call(
        _update_body,
        grid=(m // block_m,),
        in_specs=[
            pl.BlockSpec((block_m, k), lambda i: (i, 0)),
            pl.BlockSpec((block_m, k), lambda i: (i, 0)),
            pl.BlockSpec((block_m, k), lambda i: (i, 0)),
            pl.BlockSpec((k, n), lambda i: (0, 0)),
            pl.BlockSpec((k, n), lambda i: (0, 0)),
            pl.BlockSpec((1, n), lambda i: (0, 0)),
        ],
        out_specs=pl.BlockSpec((block_m, n), lambda i: (i, 0)),
        out_shape=jax.ShapeDtypeStruct((m, n), jnp.float32),
    )(h, z0, z1, w1, w2, b)


_MESH = plsc.VectorSubcoreMesh(core_axis_name="c", subcore_axis_name="s")


@functools.partial(
    pl.kernel,
    out_type=(
        jax.ShapeDtypeStruct((N_NODES, D_FEAT), jnp.float32),
        jax.ShapeDtypeStruct((N_NODES, D_FEAT), jnp.float32),
    ),
    mesh=_MESH,
    scratch_types=[
        pltpu.VMEM((E_PER_W,), jnp.int32),
        pltpu.VMEM((E_PER_W,), jnp.int32),
        pltpu.VMEM((CHUNK, D_FEAT), jnp.float32),
        pltpu.VMEM((CHUNK, D_FEAT), jnp.float32),
        pltpu.VMEM((CHUNK, D_FEAT), jnp.float32),
        pltpu.VMEM((CHUNK, D_FEAT), jnp.float32),
        pltpu.VMEM((CHUNK, D_FEAT), jnp.float32),
        pltpu.VMEM((CHUNK, D_FEAT), jnp.float32),
        pltpu.VMEM_SHARED((N_NODES, D_FEAT), jnp.float32),
        pltpu.SemaphoreType.DMA,
        pltpu.SemaphoreType.DMA,
        pltpu.SemaphoreType.DMA,
        pltpu.SemaphoreType.DMA,
        pltpu.SemaphoreType.DMA,
        pltpu.SemaphoreType.DMA,
    ],
)
def _edge_phase(hw_hbm, xew_hbm, src_hbm, dst_hbm, out0_hbm, out1_hbm,
                sidx_all, didx_all, rows0, rows1, xb0, xb1, y0, y1,
                z_sh, sg0, sg1, sx0, sx1, ss0, ss1):
    rows = (rows0, rows1)
    xb = (xb0, xb1)
    y = (y0, y1)
    sg = (sg0, sg1)
    sx = (sx0, sx1)
    ss = (ss0, ss1)

    cid = lax.axis_index("c")
    sid = lax.axis_index("s")
    wid = cid * NS + sid
    ebase = wid * E_PER_W

    pltpu.sync_copy(src_hbm.at[pl.ds(ebase, E_PER_W)], sidx_all)
    pltpu.sync_copy(dst_hbm.at[pl.ds(ebase, E_PER_W)], didx_all)

    zeros = jnp.zeros((LANES,), jnp.float32)

    def zero_row(i, carry):
        for j in range(D_FEAT // LANES):
            y0[i, pl.ds(j * LANES, LANES)] = zeros
        return carry

    lax.fori_loop(0, ZCHUNK, zero_row, 0)
    row0 = sid * STRIPE
    for k in range(NZ):
        r0 = row0 + k * ZCHUNK

        @pl.when(r0 + ZCHUNK <= N_NODES)
        def _():
            pltpu.sync_copy(y0, z_sh.at[pl.ds(r0, ZCHUNK)])

    plsc.subcore_barrier()

    pbase = wid * (E_PER_W // 2)


    def issue_gather(b, c):
        pltpu.async_copy(
            hw_hbm.at[sidx_all.at[pl.ds(c * CHUNK, CHUNK)]], rows[b], sg[b])

    def wait_gather(b, c):
        pltpu.make_async_copy(
            hw_hbm.at[sidx_all.at[pl.ds(c * CHUNK, CHUNK)]], rows[b], sg[b]).wait()

    def issue_xb(bb, p):
        pltpu.async_copy(
            xew_hbm.at[pl.ds(pbase + p * CHUNK, CHUNK)], xb[bb], sx[bb])

    def wait_xb(bb):
        pltpu.make_async_copy(
            xew_hbm.at[pl.ds(pbase, CHUNK)], xb[bb], sx[bb]).wait()

    def scatter_ref(c):
        return z_sh.at[didx_all.at[pl.ds(c * CHUNK, CHUNK)]]

    mask_hi = jnp.uint32(0xFFFF0000)

    def compute(b, xbb):
        half = b * (CHUNK // 2)

        def row_body(rp, carry):
            for h in range(2):
                r = 2 * rp + h
                for j in range(D_PACK // LANES):
                    u = jax.lax.bitcast_convert_type(
                        xb[xbb][half + rp, pl.ds(h * D_PACK + j * LANES, LANES)],
                        jnp.uint32)
                    sl_lo = pl.ds(2 * j * LANES, LANES)
                    sl_hi = pl.ds((2 * j + 1) * LANES, LANES)
                    lo = jax.lax.bitcast_convert_type(u << 16, jnp.float32)
                    hi = jax.lax.bitcast_convert_type(u & mask_hi, jnp.float32)
                    y[b][r, sl_lo] = jnp.maximum(rows[b][r, sl_lo] + lo, 0.0)
                    y[b][r, sl_hi] = jnp.maximum(rows[b][r, sl_hi] + hi, 0.0)
            return carry

        lax.fori_loop(0, CHUNK // 2, row_body, 0)

    N_PAIRS = N_CHUNKS // 2

    def do_pair(xbb, p):
        wait_xb(xbb)
        for b in range(2):
            c = 2 * p + b
            wait_gather(b, c)

            @pl.when(p > 0)
            def _():
                pltpu.make_async_copy(y[b], scatter_ref(c - 2), ss[b]).wait()

            compute(b, xbb)
            pltpu.async_copy(y[b], scatter_ref(c), ss[b], add=True)

            @pl.when(c + 2 < N_CHUNKS)
            def _():
                issue_gather(b, c + 2)

        @pl.when(p + 2 < N_PAIRS)
        def _():
            issue_xb(xbb, p + 2)

    issue_gather(0, 0)
    issue_gather(1, 1)
    issue_xb(0, 0)
    issue_xb(1, 1)

    def quad_body(q, carry):
        do_pair(0, 2 * q)
        do_pair(1, 2 * q + 1)
        return carry

    lax.fori_loop(0, N_PAIRS // 2, quad_body, 0)
    do_pair(0, N_PAIRS - 1)

    pltpu.make_async_copy(y[0], scatter_ref(N_CHUNKS - 2), ss[0]).wait()
    pltpu.make_async_copy(y[1], scatter_ref(N_CHUNKS - 1), ss[1]).wait()
    plsc.subcore_barrier()

    for k in range(NZ):
        r0 = row0 + k * ZCHUNK

        @pl.when(r0 + ZCHUNK <= N_NODES)
        def _():
            pltpu.sync_copy(z_sh.at[pl.ds(r0, ZCHUNK)], y0)

            @pl.when(cid == 0)
            def _():
                pltpu.sync_copy(y0, out0_hbm.at[pl.ds(r0, ZCHUNK)])

            @pl.when(cid == 1)
            def _():
                pltpu.sync_copy(y0, out1_hbm.at[pl.ds(r0, ZCHUNK)])


@jax.jit
def kernel(H, Xe, id_Xe, W_M, b_M, W_U, b_U):
    src = id_Xe[0].astype(jnp.int32)
    dst = id_Xe[1].astype(jnp.int32)

    plo, phi = _WPERM[:D_PACK], _WPERM[D_PACK:]
    big_perm = jnp.asarray(
        plo + [D_FEAT + c for c in plo] + phi + [D_FEAT + c for c in phi],
        dtype=jnp.int32)
    wm2 = W_M[D_FEAT:]
    w2 = jnp.zeros((2 * D_EDGE, 2 * D_FEAT), W_M.dtype)
    w2 = w2.at[:D_EDGE, :D_FEAT].set(wm2).at[D_EDGE:, D_FEAT:].set(wm2)
    w2p = w2[:, big_perm]
    b2p = jnp.concatenate([b_M, b_M])[big_perm]
    xe2 = Xe.reshape(N_EDGES // 2, 2 * D_EDGE)

    hw = _node_matmul(H, W_M[:D_FEAT], block_m=2000)
    xewb = _edge_matmul(xe2, w2p, b2p.reshape(1, -1), block_m=4000)
    z0, z1 = _edge_phase(hw, xewb, src, dst)
    return _node_update(H, z0, z1, W_U[:D_FEAT], W_U[D_FEAT:],
                        b_U.reshape(1, -1), block_m=2000)


# speedup vs baseline: 1.8189x; 1.1796x over previous
import functools

import jax
import jax.numpy as jnp
from jax import lax
from jax.experimental import pallas as pl
from jax.experimental.pallas import tpu as pltpu
from jax.experimental.pallas import tpu_sc as plsc

N_NODES = 10000
N_EDGES = 320000
D_FEAT = 128
D_EDGE = 16

NC = 2
NS = 16
LANES = 16
NW = NC * NS
E_PER_W = N_EDGES // NW
CHUNK = 40
N_CHUNKS = E_PER_W // CHUNK
STRIPE = 640
ZCHUNK = 40
NZ = STRIPE // ZCHUNK


def _mm_body(x_ref, w_ref, o_ref):
    o_ref[...] = jnp.dot(x_ref[...], w_ref[...], preferred_element_type=jnp.float32)


def _mm_bias_body(x_ref, w_ref, b_ref, o_ref):
    acc = jnp.dot(x_ref[...], w_ref[...], preferred_element_type=jnp.float32)
    o_ref[...] = acc + b_ref[...]


def _update_body(h_ref, z0_ref, z1_ref, w1_ref, w2_ref, b_ref, o_ref):
    z = z0_ref[...] + z1_ref[...]
    acc = jnp.dot(h_ref[...], w1_ref[...], preferred_element_type=jnp.float32)
    acc = acc + jnp.dot(z, w2_ref[...], preferred_element_type=jnp.float32)
    o_ref[...] = jnp.maximum(acc + b_ref[...], 0.0)


def _node_matmul(x, w, block_m):
    m, k = x.shape
    n = w.shape[1]
    return pl.pallas_call(
        _mm_body,
        grid=(m // block_m,),
        in_specs=[
            pl.BlockSpec((block_m, k), lambda i: (i, 0)),
            pl.BlockSpec((k, n), lambda i: (0, 0)),
        ],
        out_specs=pl.BlockSpec((block_m, n), lambda i: (i, 0)),
        out_shape=jax.ShapeDtypeStruct((m, n), jnp.float32),
    )(x, w)


def _edge_matmul(x, w, b, block_m):
    m, k = x.shape
    n = w.shape[1]
    return pl.pallas_call(
        _mm_bias_body,
        grid=(m // block_m,),
        in_specs=[
            pl.BlockSpec((block_m, k), lambda i: (i, 0)),
            pl.BlockSpec((k, n), lambda i: (0, 0)),
            pl.BlockSpec((1, n), lambda i: (0, 0)),
        ],
        out_specs=pl.BlockSpec((block_m, n), lambda i: (i, 0)),
        out_shape=jax.ShapeDtypeStruct((m, n), jnp.float32),
    )(x, w, b)


def _node_update(h, z0, z1, w1, w2, b, block_m):
    m, k = h.shape
    n = w1.shape[1]
    return pl.pallas_call(
        _update_body,
        grid=(m // block_m,),
        in_specs=[
            pl.BlockSpec((block_m, k), lambda i: (i, 0)),
            pl.BlockSpec((block_m, k), lambda i: (i, 0)),
            pl.BlockSpec((block_m, k), lambda i: (i, 0)),
            pl.BlockSpec((k, n), lambda i: (0, 0)),
            pl.BlockSpec((k, n), lambda i: (0, 0)),
            pl.BlockSpec((1, n), lambda i: (0, 0)),
        ],
        out_specs=pl.BlockSpec((block_m, n), lambda i: (i, 0)),
        out_shape=jax.ShapeDtypeStruct((m, n), jnp.float32),
    )(h, z0, z1, w1, w2, b)


_MESH = plsc.VectorSubcoreMesh(core_axis_name="c", subcore_axis_name="s")


@functools.partial(
    pl.kernel,
    out_type=(
        jax.ShapeDtypeStruct((N_NODES, D_FEAT), jnp.float32),
        jax.ShapeDtypeStruct((N_NODES, D_FEAT), jnp.float32),
    ),
    mesh=_MESH,
    scratch_types=[
        pltpu.VMEM((E_PER_W,), jnp.int32),
        pltpu.VMEM((E_PER_W,), jnp.int32),
        pltpu.VMEM((CHUNK, D_FEAT), jnp.float32),
        pltpu.VMEM((CHUNK, D_FEAT), jnp.float32),
        pltpu.VMEM((CHUNK, D_FEAT), jnp.float32),
        pltpu.VMEM((CHUNK, D_FEAT), jnp.float32),
        pltpu.VMEM((CHUNK, D_FEAT), jnp.float32),
        pltpu.VMEM((CHUNK, D_FEAT), jnp.float32),
        pltpu.VMEM_SHARED((N_NODES, D_FEAT), jnp.float32),
        pltpu.SemaphoreType.DMA,
        pltpu.SemaphoreType.DMA,
        pltpu.SemaphoreType.DMA,
        pltpu.SemaphoreType.DMA,
        pltpu.SemaphoreType.DMA,
        pltpu.SemaphoreType.DMA,
    ],
)
def _edge_phase(hw_hbm, xew_hbm, src_hbm, dst_hbm, out0_hbm, out1_hbm,
                sidx_all, didx_all, rows0, rows1, xb0, xb1, y0, y1,
                z_sh, sg0, sg1, sx0, sx1, ss0, ss1):
    rows = (rows0, rows1)
    xb = (xb0, xb1)
    y = (y0, y1)
    sg = (sg0, sg1)
    sx = (sx0, sx1)
    ss = (ss0, ss1)

    cid = lax.axis_index("c")
    sid = lax.axis_index("s")
    wid = cid * NS + sid
    ebase = wid * E_PER_W

    pltpu.async_copy(src_hbm.at[pl.ds(ebase, E_PER_W)], sidx_all, sg0)
    pltpu.async_copy(dst_hbm.at[pl.ds(ebase, E_PER_W)], didx_all, sg1)

    zeros = jnp.zeros((LANES,), jnp.float32)

    def zero_row(i, carry):
        for j in range(D_FEAT // LANES):
            y0[i, pl.ds(j * LANES, LANES)] = zeros
        return carry

    lax.fori_loop(0, ZCHUNK, zero_row, 0)
    row0 = sid * STRIPE
    for k in range(NZ):
        r0 = row0 + k * ZCHUNK

        @pl.when(r0 + ZCHUNK <= N_NODES)
        def _():
            pltpu.async_copy(y0, z_sh.at[pl.ds(r0, ZCHUNK)], ss0)

    for k in range(NZ):
        r0 = row0 + k * ZCHUNK

        @pl.when(r0 + ZCHUNK <= N_NODES)
        def _():
            pltpu.make_async_copy(y0, z_sh.at[pl.ds(r0, ZCHUNK)], ss0).wait()

    pltpu.make_async_copy(src_hbm.at[pl.ds(ebase, E_PER_W)], sidx_all, sg0).wait()
    pltpu.make_async_copy(dst_hbm.at[pl.ds(ebase, E_PER_W)], didx_all, sg1).wait()
    plsc.subcore_barrier()

    def issue(b, c):
        eoff = ebase + c * CHUNK
        pltpu.async_copy(xew_hbm.at[pl.ds(eoff, CHUNK)], xb[b], sx[b])
        pltpu.async_copy(
            hw_hbm.at[sidx_all.at[pl.ds(c * CHUNK, CHUNK)]], rows[b], sg[b])

    def wait_inputs(b, c):
        pltpu.make_async_copy(xew_hbm.at[pl.ds(ebase, CHUNK)], xb[b], sx[b]).wait()
        pltpu.make_async_copy(
            hw_hbm.at[sidx_all.at[pl.ds(c * CHUNK, CHUNK)]], rows[b], sg[b]).wait()

    def scatter_ref(c):
        return z_sh.at[didx_all.at[pl.ds(c * CHUNK, CHUNK)]]

    def compute(b):
        def row_body(r, carry):
            for j in range(D_FEAT // LANES):
                sl = pl.ds(j * LANES, LANES)
                y[b][r, sl] = jnp.maximum(rows[b][r, sl] + xb[b][r, sl], 0.0)
            return carry

        lax.fori_loop(0, CHUNK, row_body, 0)

    issue(0, 0)
    issue(1, 1)

    def pair_body(o, carry):
        for b in range(2):
            c = 2 * o + b
            wait_inputs(b, c)

            @pl.when(o > 0)
            def _():
                pltpu.make_async_copy(y[b], scatter_ref(c - 2), ss[b]).wait()

            compute(b)
            pltpu.async_copy(y[b], scatter_ref(c), ss[b], add=True)

            @pl.when(c + 2 < N_CHUNKS)
            def _():
                issue(b, c + 2)

        return carry

    lax.fori_loop(0, N_CHUNKS // 2, pair_body, 0)

    pltpu.make_async_copy(y[0], scatter_ref(N_CHUNKS - 2), ss[0]).wait()
    pltpu.make_async_copy(y[1], scatter_ref(N_CHUNKS - 1), ss[1]).wait()
    plsc.subcore_barrier()

    for k in range(NZ):
        r0 = row0 + k * ZCHUNK

        @pl.when(r0 + ZCHUNK <= N_NODES)
        def _():
            pltpu.sync_copy(z_sh.at[pl.ds(r0, ZCHUNK)], y0)

            @pl.when(cid == 0)
            def _():
                pltpu.sync_copy(y0, out0_hbm.at[pl.ds(r0, ZCHUNK)])

            @pl.when(cid == 1)
            def _():
                pltpu.sync_copy(y0, out1_hbm.at[pl.ds(r0, ZCHUNK)])


@jax.jit
def kernel(H, Xe, id_Xe, W_M, b_M, W_U, b_U):
    src = id_Xe[0].astype(jnp.int32)
    dst = id_Xe[1].astype(jnp.int32)
    hw = _node_matmul(H, W_M[:D_FEAT], block_m=2000)
    xewb = _edge_matmul(Xe, W_M[D_FEAT:], b_M.reshape(1, -1), block_m=4000)
    z0, z1 = _edge_phase(hw, xewb, src, dst)
    return _node_update(H, z0, z1, W_U[:D_FEAT], W_U[D_FEAT:],
                        b_U.reshape(1, -1), block_m=2000)
